# Initial kernel scaffold; baseline (speedup 1.0000x reference)
import jax
import jax.numpy as jnp
from jax.experimental import pallas as pl


def kernel(x, edge_index, edge_attr, batch, params):
    raise NotImplementedError("write your pallas kernel here")


# speedup vs baseline: 2.2709x; 2.2709x over previous
import jax
import jax.numpy as jnp
import math
from jax.experimental import pallas as pl

N = 10000
G = 128


def _tconv(x, src, dst, ea, p, pre):
    q = x @ p[pre + 'Wq'] + p[pre + 'bq']
    k = x @ p[pre + 'Wk'] + p[pre + 'bk']
    v = x @ p[pre + 'Wv'] + p[pre + 'bv']
    e = ea @ p[pre + 'We']
    kj = k[src] + e
    alpha = (q[dst] * kj).sum(-1) / math.sqrt(64.0)
    ex = jnp.exp(alpha)
    den = jax.ops.segment_sum(ex, dst, num_segments=N)
    num = jax.ops.segment_sum(ex[:, None] * (v[src] + e), dst, num_segments=N)
    agg = num / jnp.where(den > 0, den, 1.0)[:, None]
    return agg + (x @ p[pre + 'Ws'] + p[pre + 'bs'])


def _bn(x, g, b):
    return g * x / jnp.sqrt(jnp.float32(1.0 + 1e-5)) + b


def kernel(x, edge_index, edge_attr, batch, params):
    src = edge_index[0]
    dst = edge_index[1]
    c1 = _tconv(x, src, dst, edge_attr, params, 'l1_')
    h = _bn(jax.nn.relu(c1), params['g1'], params['b1'])
    c2 = _tconv(h, src, dst, edge_attr, params, 'l2_')
    h = _bn(jax.nn.relu(c2), params['g2'], params['b2'])
    sums = jax.ops.segment_sum(h, batch, num_segments=G)
    cnt = jax.ops.segment_sum(jnp.ones((h.shape[0],), h.dtype), batch, num_segments=G)
    pooled = sums / jnp.maximum(cnt, 1.0)[:, None]
    xb = _bn(pooled, params['g3'], params['b3'])
    o = jax.nn.relu(xb @ params['Wl1'] + params['bl1'])
    o = jax.nn.relu(o @ params['Wl2'] + params['bl2'])
    o = jax.nn.relu(o @ params['Wl3'] + params['bl3'])
    return o, xb, c1, c2


# speedup vs baseline: 2.6320x; 1.1590x over previous
import math

import jax
import jax.numpy as jnp
from jax import lax
from jax.experimental import pallas as pl
from jax.experimental.pallas import tpu as pltpu
from jax.experimental.pallas import tpu_sc as plsc

N = 10000
E = 320000
G = 128
C = 64

NC = 2
NS = 16
NW = NC * NS
EPW = E // NW
BLK = 80
NBLK = EPW // BLK
NGRP = BLK // 16

_BN_SCALE = 1.0 / math.sqrt(1.0 + 1e-5)


def _proj_body(x_ref, wqs, bqs, wkv, bkv, qs_ref, kv_ref):
    xv = x_ref[...]
    qs_ref[...] = jnp.dot(xv, wqs[...], preferred_element_type=jnp.float32, precision=lax.Precision.HIGHEST) + bqs[...]
    kv_ref[...] = jnp.dot(xv, wkv[...], preferred_element_type=jnp.float32, precision=lax.Precision.HIGHEST) + bkv[...]


def _proj(x, wqs, bqs, wkv, bkv):
    out = jax.ShapeDtypeStruct((N, 2 * C), jnp.float32)
    return pl.pallas_call(_proj_body, out_shape=[out, out])(x, wqs, bqs, wkv, bkv)


_EBLK = 10000


def _eproj_body(ea_ref, wb1, wb2, e1_ref, e2_ref):
    ea = ea_ref[...]
    e1_ref[...] = jnp.dot(ea, wb1[...], preferred_element_type=jnp.float32, precision=lax.Precision.HIGHEST)
    e2_ref[...] = jnp.dot(ea, wb2[...], preferred_element_type=jnp.float32, precision=lax.Precision.HIGHEST)


def _eproj(ea2, wb1, wb2):
    nsteps = (E // 2) // _EBLK
    return pl.pallas_call(
        _eproj_body,
        grid=(nsteps,),
        in_specs=[
            pl.BlockSpec((_EBLK, 32), lambda i: (i, 0)),
            pl.BlockSpec((32, 2 * C), lambda i: (0, 0)),
            pl.BlockSpec((32, 2 * C), lambda i: (0, 0)),
        ],
        out_specs=[
            pl.BlockSpec((_EBLK, 2 * C), lambda i: (i, 0)),
            pl.BlockSpec((_EBLK, 2 * C), lambda i: (i, 0)),
        ],
        out_shape=[
            jax.ShapeDtypeStruct((E // 2, 2 * C), jnp.float32),
            jax.ShapeDtypeStruct((E // 2, 2 * C), jnp.float32),
        ],
    )(ea2, wb1, wb2)


def _post1_body(acc_ref, qs_ref, g1, b1, wqs, bqs, wkv, bkv,
                c1_ref, qs2_ref, kv2_ref):
    accs = acc_ref[0] + acc_ref[1]
    num = accs[:, 0:C]
    den = accs[:, C:C + 1]
    inv = jnp.where(den > 0.0, 1.0 / den, 0.0)
    c1 = num * inv + qs_ref[...][:, C:2 * C]
    c1_ref[...] = c1
    h = jnp.maximum(c1, 0.0) * (g1[...] * _BN_SCALE) + b1[...]
    qs2_ref[...] = jnp.dot(h, wqs[...], preferred_element_type=jnp.float32, precision=lax.Precision.HIGHEST) + bqs[...]
    kv2_ref[...] = jnp.dot(h, wkv[...], preferred_element_type=jnp.float32, precision=lax.Precision.HIGHEST) + bkv[...]


def _post1(acc, qs1, g1, b1, wqs, bqs, wkv, bkv):
    return pl.pallas_call(
        _post1_body,
        out_shape=[
            jax.ShapeDtypeStruct((N, C), jnp.float32),
            jax.ShapeDtypeStruct((N, 2 * C), jnp.float32),
            jax.ShapeDtypeStruct((N, 2 * C), jnp.float32),
        ],
    )(acc, qs1, g1, b1, wqs, bqs, wkv, bkv)


def _post2_body(acc_ref, qs_ref, g2, b2, batch_ref, g3, b3,
                wl1, bl1, wl2, bl2, wl3, bl3,
                o_ref, xb_ref, c2_ref):
    accs = acc_ref[0] + acc_ref[1]
    num = accs[:, 0:C]
    den = accs[:, C:C + 1]
    inv = jnp.where(den > 0.0, 1.0 / den, 0.0)
    c2 = num * inv + qs_ref[...][:, C:2 * C]
    c2_ref[...] = c2
    h = jnp.maximum(c2, 0.0) * (g2[...] * _BN_SCALE) + b2[...]
    gids = lax.broadcasted_iota(jnp.int32, (G, N), 0)
    onehot = (gids == batch_ref[...]).astype(jnp.float32)
    sums = jnp.dot(onehot, h, preferred_element_type=jnp.float32, precision=lax.Precision.HIGHEST)
    cnt = jnp.dot(onehot, jnp.ones((N, 1), jnp.float32),
                  preferred_element_type=jnp.float32, precision=lax.Precision.HIGHEST)
    pooled = sums / jnp.maximum(cnt, 1.0)
    xb = pooled * (g3[...] * _BN_SCALE) + b3[...]
    xb_ref[...] = xb
    o = jnp.maximum(jnp.dot(xb, wl1[...], preferred_element_type=jnp.float32, precision=lax.Precision.HIGHEST) + bl1[...], 0.0)
    o = jnp.maximum(jnp.dot(o, wl2[...], preferred_element_type=jnp.float32, precision=lax.Precision.HIGHEST) + bl2[...], 0.0)
    o = jnp.maximum(jnp.dot(o, wl3[...], preferred_element_type=jnp.float32, precision=lax.Precision.HIGHEST) + bl3[...], 0.0)
    o_ref[...] = o


def _post2(acc, qs2, g2, b2, batch_row, g3, b3, wl1, bl1, wl2, bl2, wl3, bl3):
    return pl.pallas_call(
        _post2_body,
        out_shape=[
            jax.ShapeDtypeStruct((G, 2), jnp.float32),
            jax.ShapeDtypeStruct((G, C), jnp.float32),
            jax.ShapeDtypeStruct((N, C), jnp.float32),
        ],
    )(acc, qs2, g2, b2, batch_row, g3, b3, wl1, bl1, wl2, bl2, wl3, bl3)


def _sc_edge_body(qs_hbm, kv_hbm, e_hbm, src_hbm, dst_hbm, out_hbm,
                  idx_src, idx_dst, qrows, kvrows, erows, acc):
    cid = lax.axis_index("c")
    sid = lax.axis_index("s")
    w = sid * NC + cid

    zero16 = jnp.zeros((16,), jnp.float32)

    def _zrow(i, carry):
        for t in range(8):
            qrows[i, pl.ds(t * 16, 16)] = zero16
        return carry
    lax.fori_loop(0, BLK, _zrow, 0)

    def _zacc(i, carry):
        pltpu.sync_copy(qrows.at[pl.ds(0, 25)],
                        acc.at[pl.ds(sid * 625 + i * 25, 25)])
        return carry
    lax.fori_loop(0, 25, _zacc, 0)
    plsc.subcore_barrier()

    lanes = lax.iota(jnp.int32, 16)

    def _block(b, carry):
        base = w * EPW + b * BLK
        pltpu.sync_copy(src_hbm.at[pl.ds(base, BLK)], idx_src)
        pltpu.sync_copy(dst_hbm.at[pl.ds(base, BLK)], idx_dst)
        pltpu.sync_copy(kv_hbm.at[idx_src], kvrows)
        pltpu.sync_copy(qs_hbm.at[idx_dst], qrows)
        pltpu.sync_copy(e_hbm.at[pl.ds(w * (EPW // 2) + b * (BLK // 2), BLK // 2)],
                        erows)

        def _grp(g, gcarry):
            rows16 = g * 16 + lanes
            rowh = lax.shift_right_logical(rows16, 1)
            colb = (rows16 & 1) * C
            acc16 = jnp.zeros((16,), jnp.float32)
            for d in range(C):
                dcol = jnp.full((16,), d, jnp.int32)
                qc = plsc.load_gather(qrows, [rows16, dcol])
                kc = plsc.load_gather(kvrows, [rows16, dcol])
                ec = plsc.load_gather(erows, [rowh, colb + d])
                acc16 = acc16 + qc * (kc + ec)
            ex = jnp.exp(acc16 * 0.125)
            for d in range(C):
                dcol = jnp.full((16,), d, jnp.int32)
                vc = plsc.load_gather(kvrows, [rows16, dcol + C])
                ec = plsc.load_gather(erows, [rowh, colb + d])
                plsc.store_scatter(qrows, [rows16, dcol], ex * (vc + ec))
            plsc.store_scatter(qrows, [rows16, jnp.full((16,), C, jnp.int32)], ex)
            return gcarry
        lax.fori_loop(0, NGRP, _grp, 0)

        pltpu.sync_copy(qrows, acc.at[idx_dst], add=True)
        return carry
    lax.fori_loop(0, NBLK, _block, 0)

    plsc.subcore_barrier()

    @pl.when(sid == 0)
    def _():
        pltpu.sync_copy(acc, out_hbm.at[cid])


def _sc_edge(qs, kv, e, src, dst):
    mesh = plsc.VectorSubcoreMesh(core_axis_name="c", subcore_axis_name="s")
    f = pl.kernel(
        _sc_edge_body,
        out_type=jax.ShapeDtypeStruct((NC, N, 2 * C), jnp.float32),
        mesh=mesh,
        compiler_params=pltpu.CompilerParams(needs_layout_passes=False),
        scratch_types=[
            pltpu.VMEM((BLK,), jnp.int32),
            pltpu.VMEM((BLK,), jnp.int32),
            pltpu.VMEM((BLK, 2 * C), jnp.float32),
            pltpu.VMEM((BLK, 2 * C), jnp.float32),
            pltpu.VMEM((BLK // 2, 2 * C), jnp.float32),
            pltpu.VMEM_SHARED((N, 2 * C), jnp.float32),
        ],
    )
    return f(qs, kv, e, src, dst)


def kernel(x, edge_index, edge_attr, batch, params):
    p = params
    src = edge_index[0]
    dst = edge_index[1]

    def fuse_w(wa, wb):
        return jnp.concatenate([wa, wb], axis=1)

    def fuse_b(ba, bb):
        return jnp.concatenate([ba, bb]).reshape(1, 2 * C)

    def blockdiag(we):
        wb = jnp.zeros((32, 2 * C), jnp.float32)
        return wb.at[0:16, 0:C].set(we).at[16:32, C:2 * C].set(we)

    qs1, kv1 = _proj(x, fuse_w(p['l1_Wq'], p['l1_Ws']), fuse_b(p['l1_bq'], p['l1_bs']),
                     fuse_w(p['l1_Wk'], p['l1_Wv']), fuse_b(p['l1_bk'], p['l1_bv']))
    e1, e2 = _eproj(edge_attr.reshape(E // 2, 32),
                    blockdiag(p['l1_We']), blockdiag(p['l2_We']))

    acc1 = _sc_edge(qs1, kv1, e1, src, dst)
    c1, qs2, kv2 = _post1(acc1, qs1, p['g1'].reshape(1, C), p['b1'].reshape(1, C),
                          fuse_w(p['l2_Wq'], p['l2_Ws']), fuse_b(p['l2_bq'], p['l2_bs']),
                          fuse_w(p['l2_Wk'], p['l2_Wv']), fuse_b(p['l2_bk'], p['l2_bv']))

    acc2 = _sc_edge(qs2, kv2, e2, src, dst)
    o, xb, c2 = _post2(acc2, qs2, p['g2'].reshape(1, C), p['b2'].reshape(1, C),
                       batch.reshape(1, N).astype(jnp.int32),
                       p['g3'].reshape(1, C), p['b3'].reshape(1, C),
                       p['Wl1'], p['bl1'].reshape(1, C),
                       p['Wl2'], p['bl2'].reshape(1, 32),
                       p['Wl3'], p['bl3'].reshape(1, 2))
    return o, xb, c1, c2


# speedup vs baseline: 2.6483x; 1.0062x over previous
import math

import jax
import jax.numpy as jnp
from jax import lax
from jax.experimental import pallas as pl
from jax.experimental.pallas import tpu as pltpu
from jax.experimental.pallas import tpu_sc as plsc

N = 10000
E = 320000
G = 128
C = 64

NC = 2
NS = 16
NW = NC * NS
EPW = E // NW
BLK = 80
NBLK = EPW // BLK
NGRP = BLK // 16
AW = 80

_BN_SCALE = 1.0 / math.sqrt(1.0 + 1e-5)


def _proj_body(x_ref, wqs, bqs, wkv, bkv, qs_ref, kv_ref):
    xv = x_ref[...]
    qs_ref[...] = jnp.dot(xv, wqs[...], preferred_element_type=jnp.float32, precision=lax.Precision.HIGHEST) + bqs[...]
    kv_ref[...] = jnp.dot(xv, wkv[...], preferred_element_type=jnp.float32, precision=lax.Precision.HIGHEST) + bkv[...]


def _proj(x, wqs, bqs, wkv, bkv):
    out = jax.ShapeDtypeStruct((N, 2 * C), jnp.float32)
    return pl.pallas_call(_proj_body, out_shape=[out, out])(x, wqs, bqs, wkv, bkv)


_EBLK = 10000


def _eproj_body(ea_ref, wb1, wb2, e1_ref, e2_ref):
    ea = ea_ref[...]
    e1_ref[...] = jnp.dot(ea, wb1[...], preferred_element_type=jnp.float32, precision=lax.Precision.HIGHEST)
    e2_ref[...] = jnp.dot(ea, wb2[...], preferred_element_type=jnp.float32, precision=lax.Precision.HIGHEST)


def _eproj(ea2, wb1, wb2):
    nsteps = (E // 2) // _EBLK
    return pl.pallas_call(
        _eproj_body,
        grid=(nsteps,),
        in_specs=[
            pl.BlockSpec((_EBLK, 32), lambda i: (i, 0)),
            pl.BlockSpec((32, 2 * C), lambda i: (0, 0)),
            pl.BlockSpec((32, 2 * C), lambda i: (0, 0)),
        ],
        out_specs=[
            pl.BlockSpec((_EBLK, 2 * C), lambda i: (i, 0)),
            pl.BlockSpec((_EBLK, 2 * C), lambda i: (i, 0)),
        ],
        out_shape=[
            jax.ShapeDtypeStruct((E // 2, 2 * C), jnp.float32),
            jax.ShapeDtypeStruct((E // 2, 2 * C), jnp.float32),
        ],
    )(ea2, wb1, wb2)


def _post1_body(acc_ref, qs_ref, g1, b1, wqs, bqs, wkv, bkv,
                c1_ref, qs2_ref, kv2_ref):
    accs = acc_ref[0] + acc_ref[1]
    num = accs[:, 0:C]
    den = accs[:, C:C + 1]
    inv = jnp.where(den > 0.0, 1.0 / den, 0.0)
    c1 = num * inv + qs_ref[...][:, C:2 * C]
    c1_ref[...] = c1
    h = jnp.maximum(c1, 0.0) * (g1[...] * _BN_SCALE) + b1[...]
    qs2_ref[...] = jnp.dot(h, wqs[...], preferred_element_type=jnp.float32, precision=lax.Precision.HIGHEST) + bqs[...]
    kv2_ref[...] = jnp.dot(h, wkv[...], preferred_element_type=jnp.float32, precision=lax.Precision.HIGHEST) + bkv[...]


def _post1(acc, qs1, g1, b1, wqs, bqs, wkv, bkv):
    return pl.pallas_call(
        _post1_body,
        out_shape=[
            jax.ShapeDtypeStruct((N, C), jnp.float32),
            jax.ShapeDtypeStruct((N, 2 * C), jnp.float32),
            jax.ShapeDtypeStruct((N, 2 * C), jnp.float32),
        ],
    )(acc, qs1, g1, b1, wqs, bqs, wkv, bkv)


def _post2_body(acc_ref, qs_ref, g2, b2, batch_ref, g3, b3,
                wl1, bl1, wl2, bl2, wl3, bl3,
                o_ref, xb_ref, c2_ref):
    accs = acc_ref[0] + acc_ref[1]
    num = accs[:, 0:C]
    den = accs[:, C:C + 1]
    inv = jnp.where(den > 0.0, 1.0 / den, 0.0)
    c2 = num * inv + qs_ref[...][:, C:2 * C]
    c2_ref[...] = c2
    h = jnp.maximum(c2, 0.0) * (g2[...] * _BN_SCALE) + b2[...]
    gids = lax.broadcasted_iota(jnp.int32, (G, N), 0)
    onehot = (gids == batch_ref[...]).astype(jnp.float32)
    sums = jnp.dot(onehot, h, preferred_element_type=jnp.float32, precision=lax.Precision.HIGHEST)
    cnt = jnp.dot(onehot, jnp.ones((N, 1), jnp.float32),
                  preferred_element_type=jnp.float32, precision=lax.Precision.HIGHEST)
    pooled = sums / jnp.maximum(cnt, 1.0)
    xb = pooled * (g3[...] * _BN_SCALE) + b3[...]
    xb_ref[...] = xb
    o = jnp.maximum(jnp.dot(xb, wl1[...], preferred_element_type=jnp.float32, precision=lax.Precision.HIGHEST) + bl1[...], 0.0)
    o = jnp.maximum(jnp.dot(o, wl2[...], preferred_element_type=jnp.float32, precision=lax.Precision.HIGHEST) + bl2[...], 0.0)
    o = jnp.maximum(jnp.dot(o, wl3[...], preferred_element_type=jnp.float32, precision=lax.Precision.HIGHEST) + bl3[...], 0.0)
    o_ref[...] = o


def _post2(acc, qs2, g2, b2, batch_row, g3, b3, wl1, bl1, wl2, bl2, wl3, bl3):
    return pl.pallas_call(
        _post2_body,
        out_shape=[
            jax.ShapeDtypeStruct((G, 2), jnp.float32),
            jax.ShapeDtypeStruct((G, C), jnp.float32),
            jax.ShapeDtypeStruct((N, C), jnp.float32),
        ],
    )(acc, qs2, g2, b2, batch_row, g3, b3, wl1, bl1, wl2, bl2, wl3, bl3)


def _sc_edge_body(qs_hbm, kv_hbm, e_hbm, src_hbm, dst_hbm, out_hbm,
                  idx_src, idx_dst, qrows, kvrows, erows, msg, acc):
    cid = lax.axis_index("c")
    sid = lax.axis_index("s")
    w = sid * NC + cid

    zero16 = jnp.zeros((16,), jnp.float32)

    def _zrow(i, carry):
        for t in range(AW // 16):
            msg[i, pl.ds(t * 16, 16)] = zero16
        return carry
    lax.fori_loop(0, BLK, _zrow, 0)

    def _zacc(i, carry):
        pltpu.sync_copy(msg.at[pl.ds(0, 25)],
                        acc.at[pl.ds(sid * 625 + i * 25, 25)])
        return carry
    lax.fori_loop(0, 25, _zacc, 0)
    plsc.subcore_barrier()

    lanes = lax.iota(jnp.int32, 16)

    def _block(b, carry):
        base = w * EPW + b * BLK
        pltpu.sync_copy(src_hbm.at[pl.ds(base, BLK)], idx_src)
        pltpu.sync_copy(dst_hbm.at[pl.ds(base, BLK)], idx_dst)
        pltpu.sync_copy(kv_hbm.at[idx_src], kvrows)
        pltpu.sync_copy(qs_hbm.at[idx_dst], qrows)
        pltpu.sync_copy(e_hbm.at[pl.ds(w * (EPW // 2) + b * (BLK // 2), BLK // 2)],
                        erows)

        def _grp(g, gcarry):
            rows16 = g * 16 + lanes
            rowh = lax.shift_right_logical(rows16, 1)
            colb = (rows16 & 1) * C
            acc16 = jnp.zeros((16,), jnp.float32)
            for d in range(C):
                dcol = jnp.full((16,), d, jnp.int32)
                qc = plsc.load_gather(qrows, [rows16, dcol])
                kc = plsc.load_gather(kvrows, [rows16, dcol])
                ec = plsc.load_gather(erows, [rowh, colb + d])
                acc16 = acc16 + qc * (kc + ec)
            ex = jnp.exp(acc16 * 0.125)
            for d in range(C):
                dcol = jnp.full((16,), d, jnp.int32)
                vc = plsc.load_gather(kvrows, [rows16, dcol + C])
                ec = plsc.load_gather(erows, [rowh, colb + d])
                plsc.store_scatter(msg, [rows16, dcol], ex * (vc + ec))
            plsc.store_scatter(msg, [rows16, jnp.full((16,), C, jnp.int32)], ex)
            return gcarry
        lax.fori_loop(0, NGRP, _grp, 0)

        pltpu.sync_copy(msg, acc.at[idx_dst], add=True)
        return carry
    lax.fori_loop(0, NBLK, _block, 0)

    plsc.subcore_barrier()

    @pl.when(sid == 0)
    def _():
        pltpu.sync_copy(acc, out_hbm.at[cid])


def _sc_edge(qs, kv, e, src, dst):
    mesh = plsc.VectorSubcoreMesh(core_axis_name="c", subcore_axis_name="s")
    f = pl.kernel(
        _sc_edge_body,
        out_type=jax.ShapeDtypeStruct((NC, N, AW), jnp.float32),
        mesh=mesh,
        compiler_params=pltpu.CompilerParams(needs_layout_passes=False),
        scratch_types=[
            pltpu.VMEM((BLK,), jnp.int32),
            pltpu.VMEM((BLK,), jnp.int32),
            pltpu.VMEM((BLK, 2 * C), jnp.float32),
            pltpu.VMEM((BLK, 2 * C), jnp.float32),
            pltpu.VMEM((BLK // 2, 2 * C), jnp.float32),
            pltpu.VMEM((BLK, AW), jnp.float32),
            pltpu.VMEM_SHARED((N, AW), jnp.float32),
        ],
    )
    return f(qs, kv, e, src, dst)


def kernel(x, edge_index, edge_attr, batch, params):
    p = params
    src = edge_index[0]
    dst = edge_index[1]

    def fuse_w(wa, wb):
        return jnp.concatenate([wa, wb], axis=1)

    def fuse_b(ba, bb):
        return jnp.concatenate([ba, bb]).reshape(1, 2 * C)

    def blockdiag(we):
        wb = jnp.zeros((32, 2 * C), jnp.float32)
        return wb.at[0:16, 0:C].set(we).at[16:32, C:2 * C].set(we)

    qs1, kv1 = _proj(x, fuse_w(p['l1_Wq'], p['l1_Ws']), fuse_b(p['l1_bq'], p['l1_bs']),
                     fuse_w(p['l1_Wk'], p['l1_Wv']), fuse_b(p['l1_bk'], p['l1_bv']))
    e1, e2 = _eproj(edge_attr.reshape(E // 2, 32),
                    blockdiag(p['l1_We']), blockdiag(p['l2_We']))

    acc1 = _sc_edge(qs1, kv1, e1, src, dst)
    c1, qs2, kv2 = _post1(acc1, qs1, p['g1'].reshape(1, C), p['b1'].reshape(1, C),
                          fuse_w(p['l2_Wq'], p['l2_Ws']), fuse_b(p['l2_bq'], p['l2_bs']),
                          fuse_w(p['l2_Wk'], p['l2_Wv']), fuse_b(p['l2_bk'], p['l2_bv']))

    acc2 = _sc_edge(qs2, kv2, e2, src, dst)
    o, xb, c2 = _post2(acc2, qs2, p['g2'].reshape(1, C), p['b2'].reshape(1, C),
                       batch.reshape(1, N).astype(jnp.int32),
                       p['g3'].reshape(1, C), p['b3'].reshape(1, C),
                       p['Wl1'], p['bl1'].reshape(1, C),
                       p['Wl2'], p['bl2'].reshape(1, 32),
                       p['Wl3'], p['bl3'].reshape(1, 2))
    return o, xb, c1, c2
